# baseline (speedup 1.0000x reference)
import jax
import jax.numpy as jnp
from jax.experimental import pallas as pl
from jax.experimental.pallas import tpu as pltpu

NUM_TOKENS = 100000
HIDDEN_SIZE = 128
NCHUNK = 8
CHUNK = NUM_TOKENS // NCHUNK


def _dma_copy(in_ref, out_ref, sems):
    for k in range(NCHUNK):
        pltpu.make_async_copy(
            in_ref.at[pl.ds(k * CHUNK, CHUNK), :],
            out_ref.at[0, pl.ds(k * CHUNK, CHUNK), :],
            sems.at[k],
        ).start()
    for k in range(NCHUNK):
        pltpu.make_async_copy(
            in_ref.at[pl.ds(k * CHUNK, CHUNK), :],
            out_ref.at[0, pl.ds(k * CHUNK, CHUNK), :],
            sems.at[k],
        ).wait()


def kernel(embedding_weight):
    out = pl.pallas_call(
        _dma_copy,
        in_specs=[pl.BlockSpec(memory_space=pl.ANY)],
        out_specs=pl.BlockSpec(memory_space=pl.ANY),
        out_shape=jax.ShapeDtypeStruct((1, NUM_TOKENS, HIDDEN_SIZE), jnp.float32),
        scratch_shapes=[pltpu.SemaphoreType.DMA((NCHUNK,))],
    )(embedding_weight)
    return out


# speedup vs baseline: 47.2015x; 47.2015x over previous
import jax
import jax.numpy as jnp
from jax.experimental import pallas as pl

NUM_TOKENS = 100000
HIDDEN_SIZE = 128
ROWS_PER_BLOCK = 10000


def _copy_block(in_ref, out_ref):
    out_ref[0] = in_ref[...]


def kernel(embedding_weight):
    grid = (NUM_TOKENS // ROWS_PER_BLOCK,)
    out = pl.pallas_call(
        _copy_block,
        grid=grid,
        in_specs=[
            pl.BlockSpec((ROWS_PER_BLOCK, HIDDEN_SIZE), lambda i: (i, 0)),
        ],
        out_specs=pl.BlockSpec((1, ROWS_PER_BLOCK, HIDDEN_SIZE), lambda i: (0, i, 0)),
        out_shape=jax.ShapeDtypeStruct((1, NUM_TOKENS, HIDDEN_SIZE), jnp.float32),
    )(embedding_weight)
    return out


# speedup vs baseline: 49.2717x; 1.0439x over previous
import jax
import jax.numpy as jnp
from jax.experimental import pallas as pl

NUM_TOKENS = 100000
HIDDEN_SIZE = 128
ROWS_PER_BLOCK = 20000


def _copy_block(in_ref, out_ref):
    out_ref[0] = in_ref[...]


def kernel(embedding_weight):
    grid = (NUM_TOKENS // ROWS_PER_BLOCK,)
    out = pl.pallas_call(
        _copy_block,
        grid=grid,
        in_specs=[
            pl.BlockSpec((ROWS_PER_BLOCK, HIDDEN_SIZE), lambda i: (i, 0)),
        ],
        out_specs=pl.BlockSpec((1, ROWS_PER_BLOCK, HIDDEN_SIZE), lambda i: (0, i, 0)),
        out_shape=jax.ShapeDtypeStruct((1, NUM_TOKENS, HIDDEN_SIZE), jnp.float32),
    )(embedding_weight)
    return out
